# Initial kernel scaffold; baseline (speedup 1.0000x reference)
import jax
import jax.numpy as jnp
from jax.experimental import pallas as pl


def kernel(dst_embs, src_embs, edge_indices, W1, b1, W2, b2):
    raise NotImplementedError("write your pallas kernel here")


# speedup vs baseline: 2.4506x; 2.4506x over previous
import functools

import jax
import jax.numpy as jnp
from jax import lax
from jax.experimental import pallas as pl
from jax.experimental.pallas import tpu as pltpu
from jax.experimental.pallas import tpu_sc as plsc

N_NODES = 10000
N_EDGES = 320000
D = 128
H = D // 2

NC = 2
NS = 16
NW = NC * NS
EPW = N_EDGES // NW
CH = 80
NCH = EPW // CH


def _make_gather_mul():
    mesh = plsc.VectorSubcoreMesh(core_axis_name="c", subcore_axis_name="s")

    @functools.partial(
        pl.kernel,
        out_type=jax.ShapeDtypeStruct((N_EDGES, D), jnp.float32),
        mesh=mesh,
        scratch_types=[
            pltpu.VMEM((CH,), jnp.int32),
            pltpu.VMEM((CH,), jnp.int32),
            pltpu.VMEM((CH, D), jnp.float32),
            pltpu.VMEM((CH, D), jnp.float32),
            pltpu.SemaphoreType.DMA,
            pltpu.SemaphoreType.DMA,
        ],
    )
    def gather_mul(src_hbm, dst_hbm, sidx_hbm, didx_hbm, out_hbm,
                   sidx_v, didx_v, srows_v, drows_v, sem_s, sem_d):
        wid = lax.axis_index("s") * NC + lax.axis_index("c")
        base = wid * EPW

        def chunk_body(i, carry):
            off = base + i * CH
            pltpu.sync_copy(sidx_hbm.at[pl.ds(off, CH)], sidx_v)
            pltpu.sync_copy(didx_hbm.at[pl.ds(off, CH)], didx_v)
            cp_s = pltpu.async_copy(src_hbm.at[sidx_v], srows_v, sem_s)
            cp_d = pltpu.async_copy(dst_hbm.at[didx_v], drows_v, sem_d)
            cp_s.wait()
            cp_d.wait()

            def row_body(r, c2):
                for c in range(D // 16):
                    sl = pl.ds(c * 16, 16)
                    srows_v[r, sl] = srows_v[r, sl] * drows_v[r, sl]
                return c2

            lax.fori_loop(0, CH, row_body, 0)
            pltpu.sync_copy(srows_v, out_hbm.at[pl.ds(off, CH)])
            return carry

        lax.fori_loop(0, NCH, chunk_body, 0)

    return gather_mul


_gather_mul = _make_gather_mul()

BLK = 4000


def _mlp_body(x_ref, w1_ref, b1_ref, w2_ref, b2_ref, o_ref):
    x = x_ref[...]
    h = jnp.dot(x, w1_ref[...], preferred_element_type=jnp.float32)
    h = jnp.maximum(h + b1_ref[...], 0.0)
    y = jnp.dot(h, w2_ref[...], preferred_element_type=jnp.float32)
    o_ref[...] = jax.nn.sigmoid(y + b2_ref[...])


def _mlp(x, W1, b1, W2, b2):
    grid = (N_EDGES // BLK,)
    return pl.pallas_call(
        _mlp_body,
        grid=grid,
        in_specs=[
            pl.BlockSpec((BLK, D), lambda i: (i, 0)),
            pl.BlockSpec((D, H), lambda i: (0, 0)),
            pl.BlockSpec((1, H), lambda i: (0, 0)),
            pl.BlockSpec((H, 1), lambda i: (0, 0)),
            pl.BlockSpec((1, 1), lambda i: (0, 0)),
        ],
        out_specs=pl.BlockSpec((BLK, 1), lambda i: (i, 0)),
        out_shape=jax.ShapeDtypeStruct((N_EDGES, 1), jnp.float32),
    )(x, W1, b1, W2, b2)


def kernel(dst_embs, src_embs, edge_indices, W1, b1, W2, b2):
    src_idx = edge_indices[0]
    dst_idx = edge_indices[1]
    x = _gather_mul(src_embs, dst_embs, src_idx, dst_idx)
    return _mlp(x, W1, b1.reshape(1, H), W2, b2.reshape(1, 1))


# speedup vs baseline: 3.6705x; 1.4978x over previous
import functools

import jax
import jax.numpy as jnp
from jax import lax
from jax.experimental import pallas as pl
from jax.experimental.pallas import tpu as pltpu
from jax.experimental.pallas import tpu_sc as plsc

N_NODES = 10000
N_EDGES = 320000
D = 128
H = D // 2

NC = 2
NS = 16
NW = NC * NS
EPW = N_EDGES // NW
CH = 40
NCH = EPW // CH


def _make_gather_mul():
    mesh = plsc.VectorSubcoreMesh(core_axis_name="c", subcore_axis_name="s")

    @functools.partial(
        pl.kernel,
        out_type=jax.ShapeDtypeStruct((N_EDGES, D), jnp.float32),
        mesh=mesh,
        scratch_types=[
            pltpu.VMEM((EPW,), jnp.int32),
            pltpu.VMEM((EPW,), jnp.int32),
            [pltpu.VMEM((CH, D), jnp.float32) for _ in range(2)],
            [pltpu.VMEM((CH, D), jnp.float32) for _ in range(2)],
            [pltpu.VMEM((CH, D), jnp.float32) for _ in range(2)],
            [pltpu.SemaphoreType.DMA for _ in range(2)],
            [pltpu.SemaphoreType.DMA for _ in range(2)],
            [pltpu.SemaphoreType.DMA for _ in range(2)],
        ],
    )
    def gather_mul(src_hbm, dst_hbm, sidx_hbm, didx_hbm, out_hbm,
                   sidx_v, didx_v, srows, drows, orows, sem_s, sem_d, sem_o):
        wid = lax.axis_index("s") * NC + lax.axis_index("c")
        base = wid * EPW
        pltpu.sync_copy(sidx_hbm.at[pl.ds(base, EPW)], sidx_v)
        pltpu.sync_copy(didx_hbm.at[pl.ds(base, EPW)], didx_v)

        def fire_gather(c, b):
            pltpu.async_copy(src_hbm.at[sidx_v.at[pl.ds(c * CH, CH)]],
                             srows[b], sem_s[b])
            pltpu.async_copy(dst_hbm.at[didx_v.at[pl.ds(c * CH, CH)]],
                             drows[b], sem_d[b])

        def wait_gather(b):
            pltpu.make_async_copy(src_hbm.at[sidx_v.at[pl.ds(0, CH)]],
                                  srows[b], sem_s[b]).wait()
            pltpu.make_async_copy(dst_hbm.at[didx_v.at[pl.ds(0, CH)]],
                                  drows[b], sem_d[b]).wait()

        fire_gather(0, 0)
        fire_gather(1, 1)

        def pair_body(k, carry):
            for b in range(2):
                c = 2 * k + b
                wait_gather(b)

                @pl.when(c >= 2)
                def _wait_prev_out():
                    pltpu.make_async_copy(
                        orows[b], out_hbm.at[pl.ds(base, CH)], sem_o[b]).wait()

                def row_body(r, c2):
                    for j in range(D // 16):
                        sl = pl.ds(j * 16, 16)
                        orows[b][r, sl] = srows[b][r, sl] * drows[b][r, sl]
                    return c2

                lax.fori_loop(0, CH, row_body, 0)
                pltpu.async_copy(orows[b],
                                 out_hbm.at[pl.ds(base + c * CH, CH)],
                                 sem_o[b])

                @pl.when(c + 2 < NCH)
                def _prefetch():
                    fire_gather(c + 2, b)
            return carry

        lax.fori_loop(0, NCH // 2, pair_body, 0)
        for b in range(2):
            pltpu.make_async_copy(
                orows[b], out_hbm.at[pl.ds(base, CH)], sem_o[b]).wait()

    return gather_mul


_gather_mul = _make_gather_mul()

BLK = 4000


def _mlp_body(x_ref, w1_ref, b1_ref, w2_ref, b2_ref, o_ref):
    x = x_ref[...]
    h = jnp.dot(x, w1_ref[...], preferred_element_type=jnp.float32)
    h = jnp.maximum(h + b1_ref[...], 0.0)
    y = jnp.dot(h, w2_ref[...], preferred_element_type=jnp.float32)
    o_ref[...] = jax.nn.sigmoid(y + b2_ref[...])


def _mlp(x, W1, b1, W2, b2):
    grid = (N_EDGES // BLK,)
    return pl.pallas_call(
        _mlp_body,
        grid=grid,
        in_specs=[
            pl.BlockSpec((BLK, D), lambda i: (i, 0)),
            pl.BlockSpec((D, H), lambda i: (0, 0)),
            pl.BlockSpec((1, H), lambda i: (0, 0)),
            pl.BlockSpec((H, 1), lambda i: (0, 0)),
            pl.BlockSpec((1, 1), lambda i: (0, 0)),
        ],
        out_specs=pl.BlockSpec((BLK, 1), lambda i: (i, 0)),
        out_shape=jax.ShapeDtypeStruct((N_EDGES, 1), jnp.float32),
    )(x, W1, b1, W2, b2)


def kernel(dst_embs, src_embs, edge_indices, W1, b1, W2, b2):
    src_idx = edge_indices[0]
    dst_idx = edge_indices[1]
    x = _gather_mul(src_embs, dst_embs, src_idx, dst_idx)
    return _mlp(x, W1, b1.reshape(1, H), W2, b2.reshape(1, 1))
